# baseline (speedup 1.0000x reference)
import functools

import jax
import jax.numpy as jnp
from jax import lax
from jax.experimental import pallas as pl
from jax.experimental.pallas import tpu as pltpu
from jax.experimental.pallas import tpu_sc as plsc

EMB = 64
MAX_LEN = 200
BATCH = 4096

NC = 2
NS = 16
NW = NC * NS
TOTAL = BATCH * MAX_LEN
PER_W = TOTAL // NW
CHUNK = 40
NCHUNK = PER_W // CHUNK
POS_PARTS = MAX_LEN // CHUNK
LANES = 16

_mesh = plsc.VectorSubcoreMesh(core_axis_name="c", subcore_axis_name="s")


@functools.partial(
    pl.kernel,
    mesh=_mesh,
    out_type=jax.ShapeDtypeStruct((TOTAL, EMB), jnp.float32),
    compiler_params=pltpu.CompilerParams(use_tc_tiling_on_sc=False),
    scratch_types=[
        pltpu.VMEM((PER_W,), jnp.int32),
        pltpu.VMEM((MAX_LEN * EMB,), jnp.float32),
        pltpu.VMEM((CHUNK, EMB), jnp.float32),
        pltpu.VMEM((CHUNK, EMB), jnp.float32),
        pltpu.SemaphoreType.DMA,
        pltpu.SemaphoreType.DMA,
    ],
)
def _emb_lookup(tok_hbm, table_hbm, pos_hbm, out_hbm,
                idx_v, pos_v, buf0, buf1, sem0, sem1):
    wid = lax.axis_index("s") * NC + lax.axis_index("c")
    base = wid * PER_W

    pltpu.sync_copy(pos_hbm, pos_v)
    pltpu.sync_copy(tok_hbm.at[pl.ds(base, PER_W)], idx_v)

    bufs = (buf0, buf1)
    sems = (sem0, sem1)

    def gather(chunk, slot):
        idx_sl = idx_v.at[pl.ds(chunk * CHUNK, CHUNK)]
        return pltpu.make_async_copy(table_hbm.at[idx_sl], bufs[slot], sems[slot])

    def substep(chunk, slot):
        gather(chunk, slot).wait()
        buf = bufs[slot]
        pos_base = lax.rem(chunk, POS_PARTS) * (CHUNK * EMB)

        def add_row(j, carry):
            row = pos_base + j * EMB
            for c2 in range(EMB // LANES):
                buf[j, pl.ds(c2 * LANES, LANES)] += pos_v[pl.ds(row + c2 * LANES, LANES)]
            return carry

        lax.fori_loop(0, CHUNK, add_row, 0)
        pltpu.sync_copy(buf, out_hbm.at[pl.ds(base + chunk * CHUNK, CHUNK)])

        @pl.when(chunk + 2 < NCHUNK)
        def _():
            gather(chunk + 2, slot).start()

    gather(0, 0).start()
    gather(1, 1).start()

    def body(k, carry):
        substep(k * 2, 0)
        substep(k * 2 + 1, 1)
        return carry

    lax.fori_loop(0, NCHUNK // 2, body, 0)


def kernel(tokens, token_table, pos_emb):
    tok_flat = tokens.reshape(TOTAL)
    pos_flat = pos_emb.reshape(MAX_LEN * EMB)
    out = _emb_lookup(tok_flat, token_table, pos_flat)
    return out.reshape(BATCH, MAX_LEN, EMB)


# speedup vs baseline: 1.0904x; 1.0904x over previous
import functools

import jax
import jax.numpy as jnp
from jax import lax
from jax.experimental import pallas as pl
from jax.experimental.pallas import tpu as pltpu
from jax.experimental.pallas import tpu_sc as plsc

EMB = 64
MAX_LEN = 200
BATCH = 4096

NC = 2
NS = 16
NW = NC * NS
TOTAL = BATCH * MAX_LEN
PER_W = TOTAL // NW
CHUNK = 128
NCHUNK = PER_W // CHUNK
NBUF = 4
LANES = 16

_mesh = plsc.VectorSubcoreMesh(core_axis_name="c", subcore_axis_name="s")


@functools.partial(
    pl.kernel,
    mesh=_mesh,
    out_type=jax.ShapeDtypeStruct((TOTAL, EMB), jnp.float32),
    compiler_params=pltpu.CompilerParams(use_tc_tiling_on_sc=False),
    scratch_types=[
        pltpu.VMEM((PER_W,), jnp.int32),
        pltpu.VMEM((MAX_LEN * EMB,), jnp.float32),
        [pltpu.VMEM((CHUNK, EMB), jnp.float32)] * NBUF,
        [pltpu.SemaphoreType.DMA] * NBUF,
        [pltpu.SemaphoreType.DMA] * NBUF,
    ],
)
def _emb_lookup(tok_hbm, table_hbm, pos_hbm, out_hbm,
                idx_v, pos_v, bufs, gsems, ssems):
    wid = lax.axis_index("s") * NC + lax.axis_index("c")
    base = wid * PER_W

    pltpu.sync_copy(pos_hbm, pos_v)
    pltpu.sync_copy(tok_hbm.at[pl.ds(base, PER_W)], idx_v)

    def gather(chunk, slot):
        idx_sl = idx_v.at[pl.ds(chunk * CHUNK, CHUNK)]
        return pltpu.make_async_copy(table_hbm.at[idx_sl], bufs[slot], gsems[slot])

    def scatter(chunk, slot):
        dst = out_hbm.at[pl.ds(base + chunk * CHUNK, CHUNK)]
        return pltpu.make_async_copy(bufs[slot], dst, ssems[slot])

    def substep(chunk, slot):
        gather(chunk, slot).wait()
        buf = bufs[slot]
        p0 = lax.rem(chunk * CHUNK, MAX_LEN)

        def add_row(j, carry):
            r = p0 + j
            row = jnp.where(r < MAX_LEN, r, r - MAX_LEN) * EMB
            for c2 in range(EMB // LANES):
                buf[j, pl.ds(c2 * LANES, LANES)] += (
                    pos_v[pl.ds(row + c2 * LANES, LANES)])
            return carry

        lax.fori_loop(0, CHUNK, add_row, 0)

        scatter(chunk, slot).start()

        g = chunk + 2
        s2 = (slot + 2) % NBUF

        @pl.when(jnp.logical_and(chunk >= 2, g < NCHUNK))
        def _():
            scatter(chunk - 2, s2).wait()
            gather(g, s2).start()

    for s in range(NBUF):
        gather(s, s).start()

    def body(k, carry):
        for s in range(NBUF):
            substep(k * NBUF + s, s)
        return carry

    lax.fori_loop(0, NCHUNK // NBUF, body, 0)

    for s in range(NBUF):
        scatter(NCHUNK - NBUF + s, s).wait()


def kernel(tokens, token_table, pos_emb):
    tok_flat = tokens.reshape(TOTAL)
    pos_flat = pos_emb.reshape(MAX_LEN * EMB)
    out = _emb_lookup(tok_flat, token_table, pos_flat)
    return out.reshape(BATCH, MAX_LEN, EMB)


# speedup vs baseline: 1.3773x; 1.2631x over previous
import functools

import jax
import jax.numpy as jnp
from jax import lax
from jax.experimental import pallas as pl
from jax.experimental.pallas import tpu as pltpu
from jax.experimental.pallas import tpu_sc as plsc

EMB = 64
MAX_LEN = 200
BATCH = 4096

NC = 2
NS = 16
NW = NC * NS
SEQ_PER_W = BATCH // NW
NBUF = 4
LANES = 16

_mesh = plsc.VectorSubcoreMesh(core_axis_name="c", subcore_axis_name="s")


@functools.partial(
    pl.kernel,
    mesh=_mesh,
    out_type=jax.ShapeDtypeStruct((BATCH, MAX_LEN, EMB), jnp.float32),
    compiler_params=pltpu.CompilerParams(use_tc_tiling_on_sc=False),
    scratch_types=[
        pltpu.VMEM((SEQ_PER_W, MAX_LEN), jnp.int32),
        pltpu.VMEM((MAX_LEN, EMB), jnp.float32),
        [pltpu.VMEM((MAX_LEN, EMB), jnp.float32)] * NBUF,
        [pltpu.SemaphoreType.DMA] * NBUF,
        [pltpu.SemaphoreType.DMA] * NBUF,
    ],
)
def _emb_lookup(tok_hbm, table_hbm, pos_hbm, out_hbm,
                idx_v, pos_v, bufs, gsems, ssems):
    wid = lax.axis_index("s") * NC + lax.axis_index("c")
    base = wid * SEQ_PER_W

    pltpu.sync_copy(pos_hbm, pos_v)
    pltpu.sync_copy(tok_hbm.at[pl.ds(base, SEQ_PER_W)], idx_v)

    def gather(seq, slot):
        src = table_hbm.at[idx_v.at[seq]]
        return pltpu.make_async_copy(src, bufs[slot], gsems[slot])

    def scatter(seq, slot):
        return pltpu.make_async_copy(bufs[slot], out_hbm.at[base + seq],
                                     ssems[slot])

    def substep(seq, slot):
        gather(seq, slot).wait()
        buf = bufs[slot]

        def add_row(j, carry):
            for c2 in range(EMB // LANES):
                buf[j, pl.ds(c2 * LANES, LANES)] += pos_v[j, pl.ds(c2 * LANES, LANES)]
            return carry

        lax.fori_loop(0, MAX_LEN, add_row, 0)

        scatter(seq, slot).start()

        g = seq + 2
        s2 = (slot + 2) % NBUF

        @pl.when(jnp.logical_and(seq >= 2, g < SEQ_PER_W))
        def _():
            scatter(seq - 2, s2).wait()
            gather(g, s2).start()

    for s in range(NBUF):
        gather(s, s).start()

    def body(k, carry):
        for s in range(NBUF):
            substep(k * NBUF + s, s)
        return carry

    lax.fori_loop(0, SEQ_PER_W // NBUF, body, 0)

    for s in range(NBUF):
        scatter(SEQ_PER_W - NBUF + s, s).wait()


def kernel(tokens, token_table, pos_emb):
    return _emb_lookup(tokens, token_table, pos_emb)
